# initial kernel scaffold (unmeasured)
import jax
import jax.numpy as jnp
from jax import lax
from jax.experimental import pallas as pl
from jax.experimental.pallas import tpu as pltpu

N_DEV = 4


def kernel(x, w_mat):
    M, KL = x.shape
    K2, N = w_mat.shape
    MC = M // N_DEV
    NT = 4
    TN = N // NT

    def body(x_ref, w_ref, out_ref, acc_ref, rsbuf_ref,
             ta, tb, am_src, am_rcv,
             dma_sems, rs_send_sems, rs_recv_sems,
             ag_send_sems, ag_recv_sems, am_send_sems, am_recv_sems):
        my = lax.axis_index("i")
        left = (my - 1) % N_DEV
        right = (my + 1) % N_DEV

        am_rcv[...] = jnp.zeros((N_DEV, 128), jnp.float32)

        barrier_sem = pltpu.get_barrier_semaphore()
        for nbr in (left, right):
            pl.semaphore_signal(barrier_sem, inc=1, device_id=(nbr,),
                                device_id_type=pl.DeviceIdType.MESH)
        pl.semaphore_wait(barrier_sem, 2)

        def copy(src, dst, i=0):
            cp = pltpu.make_async_copy(src, dst, dma_sems.at[i])
            cp.start()
            cp.wait()

        for c in range(N_DEV):
            for t in range(NT):
                r = jnp.dot(x_ref[pl.ds(c * MC, MC), :],
                            w_ref[:, pl.ds(t * TN, TN)],
                            preferred_element_type=jnp.float32)
                ta[...] = r.astype(jnp.bfloat16)
                copy(ta, acc_ref.at[pl.ds(c * MC, MC), pl.ds(t * TN, TN)])

        amax_val = None
        for s in range(N_DEV - 1):
            c_recv = (my - 2 - s) % N_DEV
            if s == 0:
                src = acc_ref.at[pl.ds(((my - 1) % N_DEV) * MC, MC), :]
            else:
                src = rsbuf_ref.at[s - 1]
            rdma = pltpu.make_async_remote_copy(
                src_ref=src,
                dst_ref=rsbuf_ref.at[s],
                send_sem=rs_send_sems.at[s],
                recv_sem=rs_recv_sems.at[s],
                device_id=(right,),
                device_id_type=pl.DeviceIdType.MESH,
            )
            rdma.start()
            rdma.wait()
            row0 = c_recv * MC
            for t in range(NT):
                copy(rsbuf_ref.at[s, :, pl.ds(t * TN, TN)], ta, 0)
                copy(acc_ref.at[pl.ds(row0, MC), pl.ds(t * TN, TN)], tb, 1)
                v = ta[...].astype(jnp.float32) + tb[...].astype(jnp.float32)
                if s == N_DEV - 2:
                    v = jnp.maximum(v, 0.0)
                    tm = jnp.max(v)
                    amax_val = tm if amax_val is None else jnp.maximum(amax_val, tm)
                ta[...] = v.astype(jnp.bfloat16)
                copy(ta, rsbuf_ref.at[s, :, pl.ds(t * TN, TN)], 0)

        am_src[...] = jnp.full((1, 128), amax_val, jnp.float32)
        am_sends = []
        for j in range(1, N_DEV):
            p = (my + j) % N_DEV
            r = pltpu.make_async_remote_copy(
                src_ref=am_src,
                dst_ref=am_rcv.at[pl.ds(my, 1)],
                send_sem=am_send_sems.at[j],
                recv_sem=am_recv_sems.at[j],
                device_id=(p,),
                device_id_type=pl.DeviceIdType.MESH,
            )
            r.start()
            am_sends.append(r)
        for j in range(1, N_DEV):
            q = (my - j) % N_DEV
            rr = pltpu.make_async_remote_copy(
                src_ref=am_src,
                dst_ref=am_rcv.at[pl.ds(q, 1)],
                send_sem=am_send_sems.at[j],
                recv_sem=am_recv_sems.at[j],
                device_id=(q,),
                device_id_type=pl.DeviceIdType.MESH,
            )
            rr.wait_recv()
        for r in am_sends:
            r.wait_send()
        g = jnp.maximum(jnp.max(am_rcv[...]), amax_val)

        scale = g / 127.0
        inv = 127.0 / g
        last = N_DEV - 2
        for t in range(NT):
            copy(rsbuf_ref.at[last, :, pl.ds(t * TN, TN)], ta, 0)
            v = ta[...].astype(jnp.float32) * inv
            q8 = jnp.clip(jnp.round(v), -127.0, 127.0)
            tb[...] = (q8 * scale).astype(jnp.bfloat16)
            copy(tb, out_ref.at[pl.ds(my * MC, MC), pl.ds(t * TN, TN)], 1)

        for s in range(N_DEV - 1):
            c_send = (my - s) % N_DEV
            rdma = pltpu.make_async_remote_copy(
                src_ref=out_ref.at[pl.ds(c_send * MC, MC), :],
                dst_ref=out_ref.at[pl.ds(c_send * MC, MC), :],
                send_sem=ag_send_sems.at[s],
                recv_sem=ag_recv_sems.at[s],
                device_id=(right,),
                device_id_type=pl.DeviceIdType.MESH,
            )
            rdma.start()
            rdma.wait()

    return pl.pallas_call(
        body,
        out_shape=jax.ShapeDtypeStruct((M, N), jnp.bfloat16),
        in_specs=[
            pl.BlockSpec(memory_space=pltpu.MemorySpace.VMEM),
            pl.BlockSpec(memory_space=pltpu.MemorySpace.VMEM),
        ],
        out_specs=pl.BlockSpec(memory_space=pltpu.MemorySpace.HBM),
        scratch_shapes=[
            pltpu.HBM((M, N), jnp.bfloat16),
            pltpu.HBM((N_DEV - 1, MC, N), jnp.bfloat16),
            pltpu.VMEM((MC, TN), jnp.bfloat16),
            pltpu.VMEM((MC, TN), jnp.bfloat16),
            pltpu.VMEM((1, 128), jnp.float32),
            pltpu.VMEM((N_DEV, 128), jnp.float32),
            pltpu.SemaphoreType.DMA((2,)),
            pltpu.SemaphoreType.DMA((N_DEV - 1,)),
            pltpu.SemaphoreType.DMA((N_DEV - 1,)),
            pltpu.SemaphoreType.DMA((N_DEV - 1,)),
            pltpu.SemaphoreType.DMA((N_DEV - 1,)),
            pltpu.SemaphoreType.DMA((N_DEV,)),
            pltpu.SemaphoreType.DMA((N_DEV,)),
        ],
        compiler_params=pltpu.CompilerParams(collective_id=0),
    )(x, w_mat)


# baseline (device time: 1395478 ns/iter reference)
import os

import jax
import jax.numpy as jnp
from jax import lax
from jax.experimental import pallas as pl
from jax.experimental.pallas import tpu as pltpu

N_DEV = 4
PHASES = int(os.environ.get("PHASES", "5"))


def kernel(x, w_mat):
    x = x.astype(jnp.bfloat16)
    w_mat = w_mat.astype(jnp.bfloat16)
    M, KL = x.shape
    K2, N = w_mat.shape
    MC = M // N_DEV
    NT = 4
    TN = N // NT

    def body(x_ref, w_ref, out_ref, acc_ref, rsbuf_ref,
             ta, tb, am_src, am_rcv,
             dma_sems, rs_send_sems, rs_recv_sems,
             ag_send_sems, ag_recv_sems, am_send_sems, am_recv_sems):
        my = lax.axis_index("i")
        left = (my - 1) % N_DEV
        right = (my + 1) % N_DEV

        am_rcv[...] = jnp.zeros((N_DEV, 128), jnp.float32)

        barrier_sem = pltpu.get_barrier_semaphore()
        for nbr in (left, right):
            pl.semaphore_signal(barrier_sem, inc=1, device_id=(nbr,),
                                device_id_type=pl.DeviceIdType.MESH)
        pl.semaphore_wait(barrier_sem, 2)

        def copy(src, dst, i=0):
            cp = pltpu.make_async_copy(src, dst, dma_sems.at[i])
            cp.start()
            cp.wait()

        for c in range(N_DEV):
            for t in range(NT):
                r = jnp.dot(x_ref[pl.ds(c * MC, MC), :],
                            w_ref[:, pl.ds(t * TN, TN)],
                            preferred_element_type=jnp.float32)
                ta[...] = r.astype(jnp.bfloat16)
                copy(ta, acc_ref.at[pl.ds(c * MC, MC), pl.ds(t * TN, TN)])

        if PHASES < 2:
            return
        amax_val = None
        for s in range(N_DEV - 1):
            c_recv = (my - 2 - s) % N_DEV
            if s == 0:
                src = acc_ref.at[pl.ds(((my - 1) % N_DEV) * MC, MC), :]
            else:
                src = rsbuf_ref.at[s - 1]
            rdma = pltpu.make_async_remote_copy(
                src_ref=src,
                dst_ref=rsbuf_ref.at[s],
                send_sem=rs_send_sems.at[s],
                recv_sem=rs_recv_sems.at[s],
                device_id=(right,),
                device_id_type=pl.DeviceIdType.MESH,
            )
            rdma.start()
            rdma.wait()
            row0 = c_recv * MC
            for t in range(NT):
                copy(rsbuf_ref.at[s, :, pl.ds(t * TN, TN)], ta, 0)
                copy(acc_ref.at[pl.ds(row0, MC), pl.ds(t * TN, TN)], tb, 1)
                v = ta[...].astype(jnp.float32) + tb[...].astype(jnp.float32)
                if s == N_DEV - 2:
                    v = jnp.maximum(v, 0.0)
                    tm = jnp.max(v)
                    amax_val = tm if amax_val is None else jnp.maximum(amax_val, tm)
                ta[...] = v.astype(jnp.bfloat16)
                copy(ta, rsbuf_ref.at[s, :, pl.ds(t * TN, TN)], 0)

        if PHASES < 3:
            return
        am_src[...] = jnp.full((1, 128), amax_val, jnp.float32)
        for s in range(N_DEV - 1):
            src = am_src if s == 0 else am_rcv.at[pl.ds(s - 1, 1)]
            r = pltpu.make_async_remote_copy(
                src_ref=src,
                dst_ref=am_rcv.at[pl.ds(s, 1)],
                send_sem=am_send_sems.at[s],
                recv_sem=am_recv_sems.at[s],
                device_id=(right,),
                device_id_type=pl.DeviceIdType.MESH,
            )
            r.start()
            r.wait()
        g = jnp.maximum(jnp.max(am_rcv[...]), amax_val)

        if PHASES < 4:
            return
        scale = g / 127.0
        inv = 127.0 / g
        last = N_DEV - 2
        for t in range(NT):
            copy(rsbuf_ref.at[last, :, pl.ds(t * TN, TN)], ta, 0)
            v = ta[...].astype(jnp.float32) * inv
            q8 = jnp.clip(jnp.round(v), -127.0, 127.0)
            tb[...] = (q8 * scale).astype(jnp.bfloat16)
            copy(tb, out_ref.at[pl.ds(my * MC, MC), pl.ds(t * TN, TN)], 1)

        if PHASES < 5:
            return
        for s in range(N_DEV - 1):
            c_send = (my - s) % N_DEV
            rdma = pltpu.make_async_remote_copy(
                src_ref=out_ref.at[pl.ds(c_send * MC, MC), :],
                dst_ref=out_ref.at[pl.ds(c_send * MC, MC), :],
                send_sem=ag_send_sems.at[s],
                recv_sem=ag_recv_sems.at[s],
                device_id=(right,),
                device_id_type=pl.DeviceIdType.MESH,
            )
            rdma.start()
            rdma.wait()

    out, _acc, _rsbuf = pl.pallas_call(
        body,
        out_shape=[
            jax.ShapeDtypeStruct((M, N), jnp.bfloat16),
            jax.ShapeDtypeStruct((M, N), jnp.bfloat16),
            jax.ShapeDtypeStruct((N_DEV - 1, MC, N), jnp.bfloat16),
        ],
        in_specs=[
            pl.BlockSpec(memory_space=pltpu.MemorySpace.VMEM),
            pl.BlockSpec(memory_space=pltpu.MemorySpace.VMEM),
        ],
        out_specs=[
            pl.BlockSpec(memory_space=pltpu.MemorySpace.HBM),
            pl.BlockSpec(memory_space=pltpu.MemorySpace.HBM),
            pl.BlockSpec(memory_space=pltpu.MemorySpace.HBM),
        ],
        scratch_shapes=[
            pltpu.VMEM((MC, TN), jnp.bfloat16),
            pltpu.VMEM((MC, TN), jnp.bfloat16),
            pltpu.VMEM((1, 128), jnp.float32),
            pltpu.VMEM((N_DEV, 128), jnp.float32),
            pltpu.SemaphoreType.DMA((2,)),
            pltpu.SemaphoreType.DMA((N_DEV - 1,)),
            pltpu.SemaphoreType.DMA((N_DEV - 1,)),
            pltpu.SemaphoreType.DMA((N_DEV - 1,)),
            pltpu.SemaphoreType.DMA((N_DEV - 1,)),
            pltpu.SemaphoreType.DMA((N_DEV,)),
            pltpu.SemaphoreType.DMA((N_DEV,)),
        ],
        compiler_params=pltpu.CompilerParams(collective_id=0),
    )(x, w_mat)
    return out


# device time: 858821 ns/iter; 1.6249x vs baseline; 1.6249x over previous
import os

import jax
import jax.numpy as jnp
from jax import lax
from jax.experimental import pallas as pl
from jax.experimental.pallas import tpu as pltpu

N_DEV = 4
PHASES = int(os.environ.get("PHASES", "5"))


def kernel(x, w_mat):
    x = x.astype(jnp.bfloat16)
    w_mat = w_mat.astype(jnp.bfloat16)
    M, KL = x.shape
    K2, N = w_mat.shape
    MC = M // N_DEV
    NT = 4
    TN = N // NT
    NH = N // 2
    HT = NH // TN

    def body(x_ref, w_ref, out_ref, acc_ref, rbufR_ref, rbufL_ref,
             ta, tb, am_src, am_rcv,
             dma_sems, rsR_send, rsR_recv, rsL_send, rsL_recv,
             agR_send, agR_recv, agL_send, agL_recv,
             am_send_sems, am_recv_sems):
        my = lax.axis_index("i")
        left = (my - 1) % N_DEV
        right = (my + 1) % N_DEV

        am_rcv[...] = jnp.zeros((N_DEV, 128), jnp.float32)

        barrier_sem = pltpu.get_barrier_semaphore()
        for nbr in (left, right):
            pl.semaphore_signal(barrier_sem, inc=1, device_id=(nbr,),
                                device_id_type=pl.DeviceIdType.MESH)
        pl.semaphore_wait(barrier_sem, 2)

        def copy(src, dst, i=0):
            cp = pltpu.make_async_copy(src, dst, dma_sems.at[i])
            cp.start()
            cp.wait()

        for c in range(N_DEV):
            for t in range(NT):
                r = jnp.dot(x_ref[pl.ds(c * MC, MC), :],
                            w_ref[:, pl.ds(t * TN, TN)],
                            preferred_element_type=jnp.float32)
                ta[...] = r.astype(jnp.bfloat16)
                copy(ta, acc_ref.at[pl.ds(c * MC, MC), pl.ds(t * TN, TN)])

        if PHASES < 2:
            return

        amax_val = None
        for s in range(N_DEV - 1):
            srcR = (acc_ref.at[pl.ds(((my - 1) % N_DEV) * MC, MC), pl.ds(0, NH)]
                    if s == 0 else rbufR_ref.at[s - 1])
            rdmaR = pltpu.make_async_remote_copy(
                src_ref=srcR, dst_ref=rbufR_ref.at[s],
                send_sem=rsR_send.at[s], recv_sem=rsR_recv.at[s],
                device_id=(right,), device_id_type=pl.DeviceIdType.MESH)
            srcL = (acc_ref.at[pl.ds(((my + 1) % N_DEV) * MC, MC), pl.ds(NH, NH)]
                    if s == 0 else rbufL_ref.at[s - 1])
            rdmaL = pltpu.make_async_remote_copy(
                src_ref=srcL, dst_ref=rbufL_ref.at[s],
                send_sem=rsL_send.at[s], recv_sem=rsL_recv.at[s],
                device_id=(left,), device_id_type=pl.DeviceIdType.MESH)
            rdmaR.start()
            rdmaL.start()

            last_hop = s == N_DEV - 2
            cR = (my - 2 - s) % N_DEV
            cL = (my + 2 + s) % N_DEV
            rdmaR.wait()
            for t in range(HT):
                copy(rbufR_ref.at[s, :, pl.ds(t * TN, TN)], ta, 0)
                copy(acc_ref.at[pl.ds(cR * MC, MC), pl.ds(t * TN, TN)], tb, 1)
                v = ta[...].astype(jnp.float32) + tb[...].astype(jnp.float32)
                if last_hop:
                    v = jnp.maximum(v, 0.0)
                    tm = jnp.max(v)
                    amax_val = tm if amax_val is None else jnp.maximum(amax_val, tm)
                ta[...] = v.astype(jnp.bfloat16)
                copy(ta, rbufR_ref.at[s, :, pl.ds(t * TN, TN)], 0)
            rdmaL.wait()
            for t in range(HT):
                copy(rbufL_ref.at[s, :, pl.ds(t * TN, TN)], ta, 0)
                copy(acc_ref.at[pl.ds(cL * MC, MC), pl.ds(NH + t * TN, TN)], tb, 1)
                v = ta[...].astype(jnp.float32) + tb[...].astype(jnp.float32)
                if last_hop:
                    v = jnp.maximum(v, 0.0)
                    tm = jnp.max(v)
                    amax_val = jnp.maximum(amax_val, tm)
                ta[...] = v.astype(jnp.bfloat16)
                copy(ta, rbufL_ref.at[s, :, pl.ds(t * TN, TN)], 0)

        if PHASES < 3:
            return

        am_src[...] = jnp.full((1, 128), amax_val, jnp.float32)
        for s in range(N_DEV - 1):
            src = am_src if s == 0 else am_rcv.at[pl.ds(s - 1, 1)]
            r = pltpu.make_async_remote_copy(
                src_ref=src, dst_ref=am_rcv.at[pl.ds(s, 1)],
                send_sem=am_send_sems.at[s], recv_sem=am_recv_sems.at[s],
                device_id=(right,), device_id_type=pl.DeviceIdType.MESH)
            r.start()
            r.wait()
        g = jnp.maximum(jnp.max(am_rcv[...]), amax_val)

        if PHASES < 4:
            return

        scale = g / 127.0
        inv = 127.0 / g
        last = N_DEV - 2
        for t in range(NT):
            buf = rbufR_ref if t < HT else rbufL_ref
            tc = t if t < HT else t - HT
            copy(buf.at[last, :, pl.ds(tc * TN, TN)], ta, 0)
            v = ta[...].astype(jnp.float32) * inv
            q8 = jnp.clip(jnp.round(v), -127.0, 127.0)
            tb[...] = (q8 * scale).astype(jnp.bfloat16)
            copy(tb, out_ref.at[pl.ds(my * MC, MC), pl.ds(t * TN, TN)], 1)

        if PHASES < 5:
            return

        for s in range(N_DEV - 1):
            cR = ((my - s) % N_DEV) * MC
            cL = ((my + s) % N_DEV) * MC
            rdmaR = pltpu.make_async_remote_copy(
                src_ref=out_ref.at[pl.ds(cR, MC), pl.ds(0, NH)],
                dst_ref=out_ref.at[pl.ds(cR, MC), pl.ds(0, NH)],
                send_sem=agR_send.at[s], recv_sem=agR_recv.at[s],
                device_id=(right,), device_id_type=pl.DeviceIdType.MESH)
            rdmaL = pltpu.make_async_remote_copy(
                src_ref=out_ref.at[pl.ds(cL, MC), pl.ds(NH, NH)],
                dst_ref=out_ref.at[pl.ds(cL, MC), pl.ds(NH, NH)],
                send_sem=agL_send.at[s], recv_sem=agL_recv.at[s],
                device_id=(left,), device_id_type=pl.DeviceIdType.MESH)
            rdmaR.start()
            rdmaL.start()
            rdmaR.wait()
            rdmaL.wait()

    out, _acc, _rbufR, _rbufL = pl.pallas_call(
        body,
        out_shape=[
            jax.ShapeDtypeStruct((M, N), jnp.bfloat16),
            jax.ShapeDtypeStruct((M, N), jnp.bfloat16),
            jax.ShapeDtypeStruct((N_DEV - 1, MC, NH), jnp.bfloat16),
            jax.ShapeDtypeStruct((N_DEV - 1, MC, NH), jnp.bfloat16),
        ],
        in_specs=[
            pl.BlockSpec(memory_space=pltpu.MemorySpace.VMEM),
            pl.BlockSpec(memory_space=pltpu.MemorySpace.VMEM),
        ],
        out_specs=[pl.BlockSpec(memory_space=pltpu.MemorySpace.HBM)] * 4,
        scratch_shapes=[
            pltpu.VMEM((MC, TN), jnp.bfloat16),
            pltpu.VMEM((MC, TN), jnp.bfloat16),
            pltpu.VMEM((1, 128), jnp.float32),
            pltpu.VMEM((N_DEV, 128), jnp.float32),
            pltpu.SemaphoreType.DMA((2,)),
            pltpu.SemaphoreType.DMA((N_DEV - 1,)),
            pltpu.SemaphoreType.DMA((N_DEV - 1,)),
            pltpu.SemaphoreType.DMA((N_DEV - 1,)),
            pltpu.SemaphoreType.DMA((N_DEV - 1,)),
            pltpu.SemaphoreType.DMA((N_DEV - 1,)),
            pltpu.SemaphoreType.DMA((N_DEV - 1,)),
            pltpu.SemaphoreType.DMA((N_DEV - 1,)),
            pltpu.SemaphoreType.DMA((N_DEV - 1,)),
            pltpu.SemaphoreType.DMA((N_DEV - 1,)),
            pltpu.SemaphoreType.DMA((N_DEV - 1,)),
        ],
        compiler_params=pltpu.CompilerParams(collective_id=0),
    )(x, w_mat)
    return out


# device time: 682319 ns/iter; 2.0452x vs baseline; 1.2587x over previous
import os

import jax
import jax.numpy as jnp
from jax import lax
from jax.experimental import pallas as pl
from jax.experimental.pallas import tpu as pltpu

N_DEV = 4
PHASES = int(os.environ.get("PHASES", "5"))


def kernel(x, w_mat):
    x = x.astype(jnp.bfloat16)
    w_mat = w_mat.astype(jnp.bfloat16)
    M, KL = x.shape
    K2, N = w_mat.shape
    MC = M // N_DEV
    NT = 4
    TN = N // NT
    NH = N // 2
    HT = NH // TN
    NHOP = N_DEV - 1

    def body(x_ref, w_ref, out_ref, acc_ref, rbufR_ref, rbufL_ref, q8_ref,
             ta, tb, tq, am_src, am_rcv,
             dma_sems, rsR_send, rsR_recv, rsL_send, rsL_recv,
             agR_send, agR_recv, agL_send, agL_recv,
             am_send_sems, am_recv_sems):
        my = lax.axis_index("i")
        left = (my - 1) % N_DEV
        right = (my + 1) % N_DEV

        am_rcv[...] = jnp.zeros((N_DEV, 128), jnp.float32)

        barrier_sem = pltpu.get_barrier_semaphore()
        for nbr in (left, right):
            pl.semaphore_signal(barrier_sem, inc=1, device_id=(nbr,),
                                device_id_type=pl.DeviceIdType.MESH)
        pl.semaphore_wait(barrier_sem, 2)

        def copy(src, dst, i=0):
            cp = pltpu.make_async_copy(src, dst, dma_sems.at[i])
            cp.start()
            cp.wait()

        bufs = (ta, tb)
        pending = [None, None]
        for i in range(N_DEV * NT):
            c, t = divmod(i, NT)
            b = i % 2
            if pending[b] is not None:
                pending[b].wait()
            r = jnp.dot(x_ref[pl.ds(c * MC, MC), :],
                        w_ref[:, pl.ds(t * TN, TN)],
                        preferred_element_type=jnp.float32)
            bufs[b][...] = r.astype(jnp.bfloat16)
            cp = pltpu.make_async_copy(
                bufs[b], acc_ref.at[pl.ds(c * MC, MC), pl.ds(t * TN, TN)],
                dma_sems.at[b])
            cp.start()
            pending[b] = cp
        for p in pending:
            p.wait()

        if PHASES < 2:
            return

        def rs_rdma(d, s, p):
            if d == 0:
                if s == 0:
                    src = acc_ref.at[pl.ds(((my - 1) % N_DEV) * MC, MC),
                                     pl.ds(p * TN, TN)]
                else:
                    src = rbufR_ref.at[s - 1, :, pl.ds(p * TN, TN)]
                return pltpu.make_async_remote_copy(
                    src_ref=src,
                    dst_ref=rbufR_ref.at[s, :, pl.ds(p * TN, TN)],
                    send_sem=rsR_send.at[s * HT + p],
                    recv_sem=rsR_recv.at[s * HT + p],
                    device_id=(right,), device_id_type=pl.DeviceIdType.MESH)
            else:
                if s == 0:
                    src = acc_ref.at[pl.ds(((my + 1) % N_DEV) * MC, MC),
                                     pl.ds(NH + p * TN, TN)]
                else:
                    src = rbufL_ref.at[s - 1, :, pl.ds(p * TN, TN)]
                return pltpu.make_async_remote_copy(
                    src_ref=src,
                    dst_ref=rbufL_ref.at[s, :, pl.ds(p * TN, TN)],
                    send_sem=rsL_send.at[s * HT + p],
                    recv_sem=rsL_recv.at[s * HT + p],
                    device_id=(left,), device_id_type=pl.DeviceIdType.MESH)

        amax_val = None
        inflight = {}
        for p in range(HT):
            for d in (0, 1):
                r = rs_rdma(d, 0, p)
                r.start()
                inflight[(d, 0, p)] = r
        rs_sends = list(inflight.values())
        for s in range(NHOP):
            last_hop = s == NHOP - 1
            cR = (my - 2 - s) % N_DEV
            cL = (my + 2 + s) % N_DEV
            for p in range(HT):
                for d in (0, 1):
                    inflight[(d, s, p)].wait_recv()
                    rbuf = rbufR_ref if d == 0 else rbufL_ref
                    c = cR if d == 0 else cL
                    col = p * TN if d == 0 else NH + p * TN
                    copy(rbuf.at[s, :, pl.ds(p * TN, TN)], ta, 0)
                    copy(acc_ref.at[pl.ds(c * MC, MC), pl.ds(col, TN)], tb, 1)
                    v = ta[...].astype(jnp.float32) + tb[...].astype(jnp.float32)
                    if last_hop:
                        v = jnp.maximum(v, 0.0)
                        tm = jnp.max(v)
                        amax_val = tm if amax_val is None else jnp.maximum(amax_val, tm)
                    ta[...] = v.astype(jnp.bfloat16)
                    copy(ta, rbuf.at[s, :, pl.ds(p * TN, TN)], 0)
                    if not last_hop:
                        nxt = rs_rdma(d, s + 1, p)
                        nxt.start()
                        inflight[(d, s + 1, p)] = nxt
                        rs_sends.append(nxt)
        for r in rs_sends:
            r.wait_send()

        if PHASES < 3:
            return

        am_src[...] = jnp.full((1, 128), amax_val, jnp.float32)
        for s in range(NHOP):
            src = am_src if s == 0 else am_rcv.at[pl.ds(s - 1, 1)]
            r = pltpu.make_async_remote_copy(
                src_ref=src, dst_ref=am_rcv.at[pl.ds(s, 1)],
                send_sem=am_send_sems.at[s], recv_sem=am_recv_sems.at[s],
                device_id=(right,), device_id_type=pl.DeviceIdType.MESH)
            r.start()
            r.wait()
        g = jnp.maximum(jnp.max(am_rcv[...]), amax_val)

        if PHASES < 4:
            return

        scale = g / 127.0
        inv = 127.0 / g
        lastslot = NHOP - 1
        for t in range(NT):
            buf = rbufR_ref if t < HT else rbufL_ref
            tc = t if t < HT else t - HT
            copy(buf.at[lastslot, :, pl.ds(tc * TN, TN)], ta, 0)
            q8 = jnp.clip(jnp.round(ta[...].astype(jnp.float32) * inv),
                          -127.0, 127.0)
            tq[...] = q8.astype(jnp.int8)
            copy(tq, q8_ref.at[pl.ds(my * MC, MC), pl.ds(t * TN, TN)], 0)
            tb[...] = (q8 * scale).astype(jnp.bfloat16)
            copy(tb, out_ref.at[pl.ds(my * MC, MC), pl.ds(t * TN, TN)], 1)

        if PHASES < 5:
            return

        def ag_rdma(d, s):
            if d == 0:
                rows = ((my - s) % N_DEV) * MC
                cols = 0
                ssem, rsem, dev = agR_send.at[s], agR_recv.at[s], right
            else:
                rows = ((my + s) % N_DEV) * MC
                cols = NH
                ssem, rsem, dev = agL_send.at[s], agL_recv.at[s], left
            ref = q8_ref.at[pl.ds(rows, MC), pl.ds(cols, NH)]
            return pltpu.make_async_remote_copy(
                src_ref=ref, dst_ref=ref, send_sem=ssem, recv_sem=rsem,
                device_id=(dev,), device_id_type=pl.DeviceIdType.MESH)

        def dequant(rows, col0):
            for t in range(HT):
                copy(q8_ref.at[pl.ds(rows, MC), pl.ds(col0 + t * TN, TN)], tq, 0)
                tb[...] = (tq[...].astype(jnp.float32) * scale).astype(jnp.bfloat16)
                copy(tb, out_ref.at[pl.ds(rows, MC), pl.ds(col0 + t * TN, TN)], 1)

        AG_PIPE = os.environ.get("AG_PIPE", "0") == "1"
        if AG_PIPE:
            ag_inflight = {}
            for d in (0, 1):
                r = ag_rdma(d, 0)
                r.start()
                ag_inflight[(d, 0)] = r
            ag_sends = list(ag_inflight.values())
            for s in range(NHOP):
                for d in (0, 1):
                    ag_inflight[(d, s)].wait_recv()
                    if s < NHOP - 1:
                        nxt = ag_rdma(d, s + 1)
                        nxt.start()
                        ag_inflight[(d, s + 1)] = nxt
                        ag_sends.append(nxt)
                    if d == 0:
                        dequant(((my - 1 - s) % N_DEV) * MC, 0)
                    else:
                        dequant(((my + 1 + s) % N_DEV) * MC, NH)
            for r in ag_sends:
                r.wait_send()
        else:
            for s in range(NHOP):
                rR = ag_rdma(0, s)
                rL = ag_rdma(1, s)
                rR.start()
                rL.start()
                rR.wait()
                rL.wait()
                dequant(((my - 1 - s) % N_DEV) * MC, 0)
                dequant(((my + 1 + s) % N_DEV) * MC, NH)

    out, _acc, _rbufR, _rbufL, _q8 = pl.pallas_call(
        body,
        out_shape=[
            jax.ShapeDtypeStruct((M, N), jnp.bfloat16),
            jax.ShapeDtypeStruct((M, N), jnp.bfloat16),
            jax.ShapeDtypeStruct((NHOP, MC, NH), jnp.bfloat16),
            jax.ShapeDtypeStruct((NHOP, MC, NH), jnp.bfloat16),
            jax.ShapeDtypeStruct((M, N), jnp.int8),
        ],
        in_specs=[
            pl.BlockSpec(memory_space=pltpu.MemorySpace.VMEM),
            pl.BlockSpec(memory_space=pltpu.MemorySpace.VMEM),
        ],
        out_specs=[pl.BlockSpec(memory_space=pltpu.MemorySpace.HBM)] * 5,
        scratch_shapes=[
            pltpu.VMEM((MC, TN), jnp.bfloat16),
            pltpu.VMEM((MC, TN), jnp.bfloat16),
            pltpu.VMEM((MC, TN), jnp.int8),
            pltpu.VMEM((1, 128), jnp.float32),
            pltpu.VMEM((N_DEV, 128), jnp.float32),
            pltpu.SemaphoreType.DMA((2,)),
            pltpu.SemaphoreType.DMA((NHOP * 2,)),
            pltpu.SemaphoreType.DMA((NHOP * 2,)),
            pltpu.SemaphoreType.DMA((NHOP * 2,)),
            pltpu.SemaphoreType.DMA((NHOP * 2,)),
            pltpu.SemaphoreType.DMA((NHOP,)),
            pltpu.SemaphoreType.DMA((NHOP,)),
            pltpu.SemaphoreType.DMA((NHOP,)),
            pltpu.SemaphoreType.DMA((NHOP,)),
            pltpu.SemaphoreType.DMA((NHOP,)),
            pltpu.SemaphoreType.DMA((NHOP,)),
        ],
        compiler_params=pltpu.CompilerParams(
            collective_id=0, vmem_limit_bytes=100 * 1024 * 1024),
    )(x, w_mat)
    return out


# device time: 658322 ns/iter; 2.1197x vs baseline; 1.0365x over previous
import os

import jax
import jax.numpy as jnp
from jax import lax
from jax.experimental import pallas as pl
from jax.experimental.pallas import tpu as pltpu

N_DEV = 4
PHASES = int(os.environ.get("PHASES", "5"))
GEMM_OL = os.environ.get("GEMM_OL", "0") == "1"


def kernel(x, w_mat):
    x = x.astype(jnp.bfloat16)
    w_mat = w_mat.astype(jnp.bfloat16)
    M, KL = x.shape
    K2, N = w_mat.shape
    MC = M // N_DEV
    NT = 4
    TN = N // NT
    NH = N // 2
    HT = NH // TN
    NHOP = N_DEV - 1

    def body(x_ref, w_ref, out_ref, acc_ref, rbufR_ref, rbufL_ref, q8_ref,
             ta, tb, tq, am_src, am_rcv,
             dma_sems, rsR_send, rsR_recv, rsL_send, rsL_recv,
             agR_send, agR_recv, agL_send, agL_recv,
             am_send_sems, am_recv_sems):
        my = lax.axis_index("i")
        left = (my - 1) % N_DEV
        right = (my + 1) % N_DEV

        am_rcv[...] = jnp.zeros((N_DEV, 128), jnp.float32)

        barrier_sem = pltpu.get_barrier_semaphore()
        for nbr in (left, right):
            pl.semaphore_signal(barrier_sem, inc=1, device_id=(nbr,),
                                device_id_type=pl.DeviceIdType.MESH)
        pl.semaphore_wait(barrier_sem, 2)

        def copy(src, dst, i=0):
            cp = pltpu.make_async_copy(src, dst, dma_sems.at[i])
            cp.start()
            cp.wait()

        def rs_rdma(d, s, p):
            if d == 0:
                if s == 0:
                    src = acc_ref.at[pl.ds(((my - 1) % N_DEV) * MC, MC),
                                     pl.ds(p * TN, TN)]
                else:
                    src = rbufR_ref.at[s - 1, :, pl.ds(p * TN, TN)]
                return pltpu.make_async_remote_copy(
                    src_ref=src,
                    dst_ref=rbufR_ref.at[s, :, pl.ds(p * TN, TN)],
                    send_sem=rsR_send.at[s * HT + p],
                    recv_sem=rsR_recv.at[s * HT + p],
                    device_id=(right,), device_id_type=pl.DeviceIdType.MESH)
            else:
                if s == 0:
                    src = acc_ref.at[pl.ds(((my + 1) % N_DEV) * MC, MC),
                                     pl.ds(NH + p * TN, TN)]
                else:
                    src = rbufL_ref.at[s - 1, :, pl.ds(p * TN, TN)]
                return pltpu.make_async_remote_copy(
                    src_ref=src,
                    dst_ref=rbufL_ref.at[s, :, pl.ds(p * TN, TN)],
                    send_sem=rsL_send.at[s * HT + p],
                    recv_sem=rsL_recv.at[s * HT + p],
                    device_id=(left,), device_id_type=pl.DeviceIdType.MESH)

        bufs = (ta, tb)
        pending = [None, None]
        for c in range(N_DEV):
            for t in range(NT):
                b = t % 2
                if pending[b] is not None:
                    pending[b].wait()
                r = jnp.dot(x_ref[pl.ds(c * MC, MC), :],
                            w_ref[:, pl.ds(t * TN, TN)],
                            preferred_element_type=jnp.float32)
                bufs[b][...] = r.astype(jnp.bfloat16)
                cp = pltpu.make_async_copy(
                    bufs[b], acc_ref.at[pl.ds(c * MC, MC), pl.ds(t * TN, TN)],
                    dma_sems.at[b])
                cp.start()
                pending[b] = cp
            for b in (0, 1):
                if pending[b] is not None:
                    pending[b].wait()
                    pending[b] = None
            if PHASES >= 2 and GEMM_OL:
                @pl.when(c == (my - 1) % N_DEV)
                def _():
                    for p in range(HT):
                        rr = rs_rdma(0, 0, p)
                        rr.start()

                @pl.when(c == (my + 1) % N_DEV)
                def _():
                    for p in range(HT):
                        rr = rs_rdma(1, 0, p)
                        rr.start()

        if PHASES < 2:
            return

        amax_val = None
        inflight = {}
        for p in range(HT):
            for d in (0, 1):
                r = rs_rdma(d, 0, p)
                if not GEMM_OL:
                    r.start()
                inflight[(d, 0, p)] = r
        rs_sends = list(inflight.values())
        for s in range(NHOP):
            last_hop = s == NHOP - 1
            cR = (my - 2 - s) % N_DEV
            cL = (my + 2 + s) % N_DEV
            for p in range(HT):
                for d in (0, 1):
                    inflight[(d, s, p)].wait_recv()
                    rbuf = rbufR_ref if d == 0 else rbufL_ref
                    c = cR if d == 0 else cL
                    col = p * TN if d == 0 else NH + p * TN
                    copy(rbuf.at[s, :, pl.ds(p * TN, TN)], ta, 0)
                    copy(acc_ref.at[pl.ds(c * MC, MC), pl.ds(col, TN)], tb, 1)
                    v = ta[...].astype(jnp.float32) + tb[...].astype(jnp.float32)
                    if last_hop:
                        v = jnp.maximum(v, 0.0)
                        tm = jnp.max(v)
                        amax_val = tm if amax_val is None else jnp.maximum(amax_val, tm)
                    ta[...] = v.astype(jnp.bfloat16)
                    copy(ta, rbuf.at[s, :, pl.ds(p * TN, TN)], 0)
                    if not last_hop:
                        nxt = rs_rdma(d, s + 1, p)
                        nxt.start()
                        inflight[(d, s + 1, p)] = nxt
                        rs_sends.append(nxt)
        for r in rs_sends:
            r.wait_send()

        if PHASES < 3:
            return

        am_src[...] = jnp.full((1, 128), amax_val, jnp.float32)
        for s in range(NHOP):
            src = am_src if s == 0 else am_rcv.at[pl.ds(s - 1, 1)]
            r = pltpu.make_async_remote_copy(
                src_ref=src, dst_ref=am_rcv.at[pl.ds(s, 1)],
                send_sem=am_send_sems.at[s], recv_sem=am_recv_sems.at[s],
                device_id=(right,), device_id_type=pl.DeviceIdType.MESH)
            r.start()
            r.wait()
        g = jnp.maximum(jnp.max(am_rcv[...]), amax_val)

        if PHASES < 4:
            return

        scale = g / 127.0
        inv = 127.0 / g
        lastslot = NHOP - 1
        for t in range(NT):
            buf = rbufR_ref if t < HT else rbufL_ref
            tc = t if t < HT else t - HT
            copy(buf.at[lastslot, :, pl.ds(tc * TN, TN)], ta, 0)
            q8 = jnp.clip(jnp.round(ta[...].astype(jnp.float32) * inv),
                          -127.0, 127.0)
            tq[...] = q8.astype(jnp.int8)
            copy(tq, q8_ref.at[pl.ds(my * MC, MC), pl.ds(t * TN, TN)], 0)
            tb[...] = (q8 * scale).astype(jnp.bfloat16)
            copy(tb, out_ref.at[pl.ds(my * MC, MC), pl.ds(t * TN, TN)], 1)

        if PHASES < 5:
            return

        def ag_rdma(d, s):
            if d == 0:
                rows = ((my - s) % N_DEV) * MC
                cols = 0
                ssem, rsem, dev = agR_send.at[s], agR_recv.at[s], right
            else:
                rows = ((my + s) % N_DEV) * MC
                cols = NH
                ssem, rsem, dev = agL_send.at[s], agL_recv.at[s], left
            ref = q8_ref.at[pl.ds(rows, MC), pl.ds(cols, NH)]
            return pltpu.make_async_remote_copy(
                src_ref=ref, dst_ref=ref, send_sem=ssem, recv_sem=rsem,
                device_id=(dev,), device_id_type=pl.DeviceIdType.MESH)

        def dequant(rows, col0):
            for t in range(HT):
                copy(q8_ref.at[pl.ds(rows, MC), pl.ds(col0 + t * TN, TN)], tq, 0)
                tb[...] = (tq[...].astype(jnp.float32) * scale).astype(jnp.bfloat16)
                copy(tb, out_ref.at[pl.ds(rows, MC), pl.ds(col0 + t * TN, TN)], 1)

        AG_PIPE = os.environ.get("AG_PIPE", "0") == "1"
        if AG_PIPE:
            ag_inflight = {}
            for d in (0, 1):
                r = ag_rdma(d, 0)
                r.start()
                ag_inflight[(d, 0)] = r
            ag_sends = list(ag_inflight.values())
            for s in range(NHOP):
                for d in (0, 1):
                    ag_inflight[(d, s)].wait_recv()
                    if s < NHOP - 1:
                        nxt = ag_rdma(d, s + 1)
                        nxt.start()
                        ag_inflight[(d, s + 1)] = nxt
                        ag_sends.append(nxt)
                    if d == 0:
                        dequant(((my - 1 - s) % N_DEV) * MC, 0)
                    else:
                        dequant(((my + 1 + s) % N_DEV) * MC, NH)
            for r in ag_sends:
                r.wait_send()
        else:
            cur = [ag_rdma(0, 0), ag_rdma(1, 0)]
            cur[0].start()
            cur[1].start()
            for s in range(NHOP):
                cur[0].wait()
                cur[1].wait()
                if s < NHOP - 1:
                    nxt = [ag_rdma(0, s + 1), ag_rdma(1, s + 1)]
                    nxt[0].start()
                    nxt[1].start()
                dequant(((my - 1 - s) % N_DEV) * MC, 0)
                dequant(((my + 1 + s) % N_DEV) * MC, NH)
                if s < NHOP - 1:
                    cur = nxt

    out, _acc, _rbufR, _rbufL, _q8 = pl.pallas_call(
        body,
        out_shape=[
            jax.ShapeDtypeStruct((M, N), jnp.bfloat16),
            jax.ShapeDtypeStruct((M, N), jnp.bfloat16),
            jax.ShapeDtypeStruct((NHOP, MC, NH), jnp.bfloat16),
            jax.ShapeDtypeStruct((NHOP, MC, NH), jnp.bfloat16),
            jax.ShapeDtypeStruct((M, N), jnp.int8),
        ],
        in_specs=[
            pl.BlockSpec(memory_space=pltpu.MemorySpace.VMEM),
            pl.BlockSpec(memory_space=pltpu.MemorySpace.VMEM),
        ],
        out_specs=[pl.BlockSpec(memory_space=pltpu.MemorySpace.HBM)] * 5,
        scratch_shapes=[
            pltpu.VMEM((MC, TN), jnp.bfloat16),
            pltpu.VMEM((MC, TN), jnp.bfloat16),
            pltpu.VMEM((MC, TN), jnp.int8),
            pltpu.VMEM((1, 128), jnp.float32),
            pltpu.VMEM((N_DEV, 128), jnp.float32),
            pltpu.SemaphoreType.DMA((2,)),
            pltpu.SemaphoreType.DMA((NHOP * 2,)),
            pltpu.SemaphoreType.DMA((NHOP * 2,)),
            pltpu.SemaphoreType.DMA((NHOP * 2,)),
            pltpu.SemaphoreType.DMA((NHOP * 2,)),
            pltpu.SemaphoreType.DMA((NHOP,)),
            pltpu.SemaphoreType.DMA((NHOP,)),
            pltpu.SemaphoreType.DMA((NHOP,)),
            pltpu.SemaphoreType.DMA((NHOP,)),
            pltpu.SemaphoreType.DMA((NHOP,)),
            pltpu.SemaphoreType.DMA((NHOP,)),
        ],
        compiler_params=pltpu.CompilerParams(
            collective_id=0, vmem_limit_bytes=100 * 1024 * 1024),
    )(x, w_mat)
    return out
